# baseline (speedup 1.0000x reference)
import functools

import jax
import jax.numpy as jnp
from jax import lax
from jax.experimental import pallas as pl
from jax.experimental.pallas import tpu as pltpu
from jax.experimental.pallas import tpu_sc as plsc

FACTORS = 64
L = 16
NC = 2
NS = 16
NW = NC * NS
IDX_CHUNK = 128
WB_PAD = 96


def _bcast_lane(vec, lane):
    idx = jnp.full((L, 1), lane, jnp.int32)
    dn = lax.GatherDimensionNumbers(
        offset_dims=(), collapsed_slice_dims=(0,), start_index_map=(0,))
    return lax.gather(vec, idx, dn, (1,),
                      mode=lax.GatherScatterMode.PROMISE_IN_BOUNDS)


def _sc_forward(uidx, iidx, utab, itab, wb):
    n_chunks = uidx.shape[1]
    b_per_w = n_chunks * IDX_CHUNK
    batch = NW * b_per_w
    groups_per_chunk = IDX_CHUNK // L
    mesh = plsc.VectorSubcoreMesh(core_axis_name="c", subcore_axis_name="s")

    @functools.partial(
        pl.kernel,
        mesh=mesh,
        out_type=jax.ShapeDtypeStruct((batch,), jnp.float32),
        compiler_params=pltpu.CompilerParams(
            needs_layout_passes=False, use_tc_tiling_on_sc=False),
        scratch_types=[
            pltpu.VMEM((n_chunks, IDX_CHUNK), jnp.int32),
            pltpu.VMEM((n_chunks, IDX_CHUNK), jnp.int32),
            pltpu.VMEM((b_per_w, FACTORS), jnp.float32),
            pltpu.VMEM((b_per_w, FACTORS), jnp.float32),
            pltpu.VMEM((WB_PAD,), jnp.float32),
            pltpu.VMEM((b_per_w,), jnp.float32),
            pltpu.SemaphoreType.DMA,
        ],
    )
    def k(uidx_hbm, iidx_hbm, utab_hbm, itab_hbm, wb_hbm, out_hbm,
          uidx_v, iidx_v, urows_v, irows_v, wb_v, out_v, sem):
        wid = lax.axis_index("s") * NC + lax.axis_index("c")
        pltpu.sync_copy(uidx_hbm.at[wid], uidx_v)
        pltpu.sync_copy(iidx_hbm.at[wid], iidx_v)
        pltpu.sync_copy(wb_hbm, wb_v)

        copies = []
        for j in range(n_chunks):
            copies.append((
                pltpu.async_copy(
                    utab_hbm.at[uidx_v.at[j]],
                    urows_v.at[pl.ds(j * IDX_CHUNK, IDX_CHUNK)], sem),
                pltpu.async_copy(
                    itab_hbm.at[iidx_v.at[j]],
                    irows_v.at[pl.ds(j * IDX_CHUNK, IDX_CHUNK)], sem),
            ))

        w = [wb_v[pl.ds(q * L, L)] for q in range(FACTORS // L)]
        bias = _bcast_lane(wb_v[pl.ds(FACTORS, L)], 0)
        lane_iota = lax.iota(jnp.int32, L)
        zeros = jnp.zeros((L,), jnp.float32)

        for j in range(n_chunks):
            copies[j][0].wait()
            copies[j][1].wait()
            chunk_base = j * IDX_CHUNK

            def group_body(g, _, chunk_base=chunk_base):
                res = zeros
                for r in range(L):
                    i = chunk_base + g * L + r
                    acc = (urows_v[i, pl.ds(0, L)]
                           * irows_v[i, pl.ds(0, L)]) * w[0]
                    for q in range(1, FACTORS // L):
                        acc += (urows_v[i, pl.ds(q * L, L)]
                                * irows_v[i, pl.ds(q * L, L)]) * w[q]
                    total = _bcast_lane(plsc.cumsum(acc), L - 1)
                    res = jnp.where(lane_iota == r, total, res)
                res = jnp.maximum(res + bias, 0.0)
                out_v[pl.ds(chunk_base + g * L, L)] = res
                return 0

            lax.fori_loop(0, groups_per_chunk, group_body, 0)

        pltpu.sync_copy(out_v, out_hbm.at[pl.ds(wid * b_per_w, b_per_w)])

    return k(uidx, iidx, utab, itab, wb)


def kernel(user_coordinates, item_coordinates, user_table, item_table, W1, b1):
    batch = user_coordinates.shape[0]
    uidx = user_coordinates.astype(jnp.int32).reshape(NW, -1, IDX_CHUNK)
    iidx = item_coordinates.astype(jnp.int32).reshape(NW, -1, IDX_CHUNK)
    wb = jnp.concatenate(
        [W1.reshape(-1), b1.reshape(-1),
         jnp.zeros((WB_PAD - FACTORS - 1,), jnp.float32)])
    out = _sc_forward(uidx, iidx, user_table, item_table, wb)
    return out.reshape(batch, 1)


# speedup vs baseline: 1.9600x; 1.9600x over previous
import functools

import jax
import jax.numpy as jnp
from jax import lax
from jax.experimental import pallas as pl
from jax.experimental.pallas import tpu as pltpu
from jax.experimental.pallas import tpu_sc as plsc

FACTORS = 64
L = 16
NC = 2
NS = 16
NW = NC * NS
SLAB = 128
BATCH = 16384
ITEMS_PER_TILE = BATCH // NS
GROUPS = ITEMS_PER_TILE // SLAB
ROWS_PAD = 128
WB_PAD = 96


def _bcast_lane0(vec):
    idx = jnp.zeros((L, 1), jnp.int32)
    dn = lax.GatherDimensionNumbers(
        offset_dims=(), collapsed_slice_dims=(0,), start_index_map=(0,))
    return lax.gather(vec, idx, dn, (1,),
                      mode=lax.GatherScatterMode.PROMISE_IN_BOUNDS)


def _bcast_dyn(vec, lane):
    idx = jnp.full((L, 1), lane, jnp.int32)
    dn = lax.GatherDimensionNumbers(
        offset_dims=(), collapsed_slice_dims=(0,), start_index_map=(0,))
    return lax.gather(vec, idx, dn, (1,),
                      mode=lax.GatherScatterMode.PROMISE_IN_BOUNDS)


def _gather_phase(su_u, pu_u, si_i, pi_i, utab_t, itab_t):
    mesh = plsc.VectorSubcoreMesh(core_axis_name="c", subcore_axis_name="s")

    @functools.partial(
        pl.kernel,
        mesh=mesh,
        out_type=(
            jax.ShapeDtypeStruct((BATCH, ROWS_PAD), jnp.float32),
            jax.ShapeDtypeStruct((BATCH, ROWS_PAD), jnp.float32),
        ),
        scratch_types=[
            pltpu.VMEM((GROUPS, SLAB), jnp.int32),
            pltpu.VMEM((GROUPS, SLAB), jnp.int32),
            pltpu.VMEM((FACTORS, SLAB), jnp.float32),
            pltpu.VMEM((SLAB, ROWS_PAD), jnp.float32),
            pltpu.SemaphoreType.DMA,
        ],
        compiler_params=pltpu.CompilerParams(
            needs_layout_passes=False, use_tc_tiling_on_sc=True),
    )
    def k(su_ref, pu_ref, si_ref, pi_ref, ut_ref, it_ref, u_out, v_out,
          srt_v, pos_v, slab_v, ext_v, sem):
        c = lax.axis_index("c")
        s = lax.axis_index("s")
        lane_iota = lax.iota(jnp.int32, L)

        def side(tab, srt_hbm, pos_hbm, out_hbm):
            pltpu.sync_copy(srt_hbm.at[s], srt_v)
            pltpu.sync_copy(pos_hbm.at[s], pos_v)
            prev = jnp.int32(-1)
            for g in range(GROUPS):
                def body(i, prev, g=g):
                    chunk_base = (i >> 4) << 4
                    chunk = srt_v[g, pl.ds(pl.multiple_of(chunk_base, 8), L)]
                    j = i & 15
                    clv = _bcast_dyn(chunk & (SLAB - 1), j)
                    ctv = _bcast_dyn(lax.shift_right_logical(chunk, 7), j)
                    ct = ctv[0]

                    @pl.when(ct != prev)
                    def _():
                        off = pl.multiple_of(ct * SLAB, SLAB)
                        pltpu.sync_copy(tab.at[:, pl.ds(off, SLAB)], slab_v)

                    for q in range(FACTORS // L):
                        vec = plsc.load_gather(
                            slab_v, [lane_iota + q * L, clv])
                        ext_v[i, pl.ds(q * L, L)] = vec
                    return ct

                prev = lax.fori_loop(0, SLAB, body, prev)
                pltpu.async_copy(ext_v, out_hbm.at[pos_v.at[g]], sem).wait()

        @pl.when(c == 0)
        def _():
            side(ut_ref, su_ref, pu_ref, u_out)

        @pl.when(c == 1)
        def _():
            side(it_ref, si_ref, pi_ref, v_out)

    return k(su_u, pu_u, si_i, pi_i, utab_t, itab_t)


def _dot_phase(u_rows, v_rows, wb):
    mesh = plsc.VectorSubcoreMesh(core_axis_name="c", subcore_axis_name="s")
    rows_per_w = BATCH // NW
    n_chunks = rows_per_w // SLAB

    @functools.partial(
        pl.kernel,
        mesh=mesh,
        out_type=jax.ShapeDtypeStruct((NW, rows_per_w), jnp.float32),
        scratch_types=[
            pltpu.VMEM((SLAB, ROWS_PAD), jnp.float32),
            pltpu.VMEM((SLAB, ROWS_PAD), jnp.float32),
            pltpu.VMEM((WB_PAD,), jnp.float32),
            pltpu.VMEM((rows_per_w,), jnp.float32),
        ],
        compiler_params=pltpu.CompilerParams(
            needs_layout_passes=False, use_tc_tiling_on_sc=True),
    )
    def k(u_hbm, v_hbm, wb_hbm, out_hbm, u_v, v_v, wb_v, out_v):
        wid = lax.axis_index("s") * NC + lax.axis_index("c")
        pltpu.sync_copy(wb_hbm, wb_v)
        w = [wb_v[pl.ds(q * L, L)] for q in range(FACTORS // L)]
        bias = _bcast_lane0(wb_v[pl.ds(FACTORS, L)])
        lane_iota = lax.iota(jnp.int32, L)
        zeros = jnp.zeros((L,), jnp.float32)

        for cc in range(n_chunks):
            row0 = pl.multiple_of(wid * rows_per_w + cc * SLAB, 8)
            pltpu.sync_copy(u_hbm.at[pl.ds(row0, SLAB)], u_v)
            pltpu.sync_copy(v_hbm.at[pl.ds(row0, SLAB)], v_v)

            def group_body(g, _, cc=cc):
                res = zeros
                for r in range(L):
                    i = g * L + r
                    acc = (u_v[i, pl.ds(0, L)] * v_v[i, pl.ds(0, L)]) * w[0]
                    for q in range(1, FACTORS // L):
                        acc += (u_v[i, pl.ds(q * L, L)]
                                * v_v[i, pl.ds(q * L, L)]) * w[q]
                    cum = plsc.cumsum(acc)
                    total = _bcast_dyn(cum, L - 1)
                    res = jnp.where(lane_iota == r, total, res)
                res = jnp.maximum(res + bias, 0.0)
                out_v[pl.ds(cc * SLAB + g * L, L)] = res
                return 0

            lax.fori_loop(0, SLAB // L, group_body, 0)

        pltpu.sync_copy(out_v, out_hbm.at[wid])

    return k(u_rows, v_rows, wb)


def kernel(user_coordinates, item_coordinates, user_table, item_table, W1, b1):
    batch = user_coordinates.shape[0]
    uidx = user_coordinates.astype(jnp.int32)
    iidx = item_coordinates.astype(jnp.int32)
    su = jnp.sort(uidx).reshape(NS, GROUPS, SLAB)
    pu = jnp.argsort(uidx).astype(jnp.int32).reshape(NS, GROUPS, SLAB)
    si = jnp.sort(iidx).reshape(NS, GROUPS, SLAB)
    pi = jnp.argsort(iidx).astype(jnp.int32).reshape(NS, GROUPS, SLAB)
    wb = jnp.concatenate(
        [W1.reshape(-1), b1.reshape(-1),
         jnp.zeros((WB_PAD - FACTORS - 1,), jnp.float32)])
    u_rows, v_rows = _gather_phase(
        su, pu, si, pi, user_table.T, item_table.T)
    out = _dot_phase(u_rows, v_rows, wb)
    return out.reshape(batch, 1)


# speedup vs baseline: 2.8597x; 1.4591x over previous
import functools

import jax
import jax.numpy as jnp
from jax import lax
from jax.experimental import pallas as pl
from jax.experimental.pallas import tpu as pltpu
from jax.experimental.pallas import tpu_sc as plsc

FACTORS = 64
L = 16
NC = 2
NS = 16
NW = NC * NS
SLAB = 128
BATCH = 16384
ITEMS_PER_TILE = BATCH // NS
GROUPS = ITEMS_PER_TILE // SLAB
ROWS_PAD = 128
WB_PAD = 96


def _bcast_lane0(vec):
    idx = jnp.zeros((L, 1), jnp.int32)
    dn = lax.GatherDimensionNumbers(
        offset_dims=(), collapsed_slice_dims=(0,), start_index_map=(0,))
    return lax.gather(vec, idx, dn, (1,),
                      mode=lax.GatherScatterMode.PROMISE_IN_BOUNDS)


def _bcast_dyn(vec, lane):
    idx = jnp.full((L, 1), lane, jnp.int32)
    dn = lax.GatherDimensionNumbers(
        offset_dims=(), collapsed_slice_dims=(0,), start_index_map=(0,))
    return lax.gather(vec, idx, dn, (1,),
                      mode=lax.GatherScatterMode.PROMISE_IN_BOUNDS)


NBUF = 4
LOOKAHEAD = 3


def _gather_phase(su_u, pu_u, ou_u, lu_u, si_i, pi_i, oi_i, li_i,
                  utab_t, itab_t):
    mesh = plsc.VectorSubcoreMesh(core_axis_name="c", subcore_axis_name="s")

    @functools.partial(
        pl.kernel,
        mesh=mesh,
        out_type=(
            jax.ShapeDtypeStruct((BATCH, ROWS_PAD), jnp.float32),
            jax.ShapeDtypeStruct((BATCH, ROWS_PAD), jnp.float32),
        ),
        scratch_types=[
            pltpu.VMEM((GROUPS, SLAB), jnp.int32),
            pltpu.VMEM((GROUPS, SLAB), jnp.int32),
            pltpu.VMEM((GROUPS, SLAB), jnp.int32),
            pltpu.VMEM((GROUPS, SLAB), jnp.int32),
            pltpu.VMEM((NBUF, FACTORS, SLAB), jnp.float32),
            pltpu.VMEM((SLAB, ROWS_PAD), jnp.float32),
            pltpu.SemaphoreType.DMA,
            pltpu.SemaphoreType.DMA,
            pltpu.SemaphoreType.DMA,
            pltpu.SemaphoreType.DMA,
            pltpu.SemaphoreType.DMA,
        ],
        compiler_params=pltpu.CompilerParams(
            needs_layout_passes=False, use_tc_tiling_on_sc=True),
    )
    def k(su_ref, pu_ref, ou_ref, lu_ref, si_ref, pi_ref, oi_ref, li_ref,
          ut_ref, it_ref, u_out, v_out,
          srt_v, pos_v, ord_v, slabs_v, ring_v, ext_v,
          sem0, sem1, sem2, sem3, scat_sem):
        c = lax.axis_index("c")
        s = lax.axis_index("s")
        sems = [sem0, sem1, sem2, sem3]
        lane_iota = lax.iota(jnp.int32, L)

        def side(tab, srt_hbm, pos_hbm, ordh, slabh, out_hbm):
            pltpu.sync_copy(srt_hbm.at[s], srt_v)
            pltpu.sync_copy(pos_hbm.at[s], pos_v)
            pltpu.sync_copy(ordh.at[s], ord_v)
            pltpu.sync_copy(slabh.at[s], slabs_v)

            def fire(b, p):
                pc = jnp.minimum(p, GROUPS * SLAB - 1)
                prow = pc >> 7
                pcb = ((pc & 127) >> 4) << 4
                pchunk = slabs_v[prow, pl.ds(pl.multiple_of(pcb, 8), L)]
                sid = _bcast_dyn(pchunk, pc & 15)[0]
                off = pl.multiple_of(sid * SLAB, SLAB)
                pltpu.async_copy(tab.at[:, pl.ds(off, SLAB)],
                                 ring_v.at[b], sems[b])

            def drain(b):
                pltpu.make_async_copy(tab.at[:, pl.ds(0, SLAB)],
                                      ring_v.at[b], sems[b]).wait()

            for b in range(LOOKAHEAD):
                fire(b, jnp.int32(b))

            prev = jnp.int32(-1)
            for g in range(GROUPS):
                def body(i, prev, g=g):
                    chunk_base = (i >> 4) << 4
                    chunk = srt_v[g, pl.ds(pl.multiple_of(chunk_base, 8), L)]
                    j = i & 15
                    clv = _bcast_dyn(chunk & (SLAB - 1), j)
                    ochunk = ord_v[g, pl.ds(pl.multiple_of(chunk_base, 8), L)]
                    odv = _bcast_dyn(ochunk, j)
                    od = odv[0]

                    @pl.when(od != prev)
                    def _():
                        for b in range(NBUF):
                            @pl.when((od & 3) == b)
                            def _(b=b):
                                drain(b)
                                fire((b + LOOKAHEAD) % NBUF,
                                     od + LOOKAHEAD)

                    bsel = odv & 3
                    for q in range(FACTORS // L):
                        vec = plsc.load_gather(
                            ring_v, [bsel, lane_iota + q * L, clv])
                        ext_v[i, pl.ds(q * L, L)] = vec
                    return od

                prev = lax.fori_loop(0, SLAB, body, prev)
                pltpu.async_copy(ext_v, out_hbm.at[pos_v.at[g]],
                                 scat_sem).wait()

            for r in range(NBUF):
                @pl.when((prev & 3) == r)
                def _(r=r):
                    for d in range(1, LOOKAHEAD + 1):
                        drain((r + d) % NBUF)

        @pl.when(c == 0)
        def _():
            side(ut_ref, su_ref, pu_ref, ou_ref, lu_ref, u_out)

        @pl.when(c == 1)
        def _():
            side(it_ref, si_ref, pi_ref, oi_ref, li_ref, v_out)

    return k(su_u, pu_u, ou_u, lu_u, si_i, pi_i, oi_i, li_i, utab_t, itab_t)


def _dot_phase(u_rows, v_rows, wb):
    mesh = plsc.VectorSubcoreMesh(core_axis_name="c", subcore_axis_name="s")
    rows_per_w = BATCH // NW
    n_chunks = rows_per_w // SLAB

    @functools.partial(
        pl.kernel,
        mesh=mesh,
        out_type=jax.ShapeDtypeStruct((NW, rows_per_w), jnp.float32),
        scratch_types=[
            pltpu.VMEM((SLAB, ROWS_PAD), jnp.float32),
            pltpu.VMEM((SLAB, ROWS_PAD), jnp.float32),
            pltpu.VMEM((WB_PAD,), jnp.float32),
            pltpu.VMEM((rows_per_w,), jnp.float32),
        ],
        compiler_params=pltpu.CompilerParams(
            needs_layout_passes=False, use_tc_tiling_on_sc=True),
    )
    def k(u_hbm, v_hbm, wb_hbm, out_hbm, u_v, v_v, wb_v, out_v):
        wid = lax.axis_index("s") * NC + lax.axis_index("c")
        pltpu.sync_copy(wb_hbm, wb_v)
        w = [wb_v[pl.ds(q * L, L)] for q in range(FACTORS // L)]
        bias = _bcast_lane0(wb_v[pl.ds(FACTORS, L)])
        lane_iota = lax.iota(jnp.int32, L)
        zeros = jnp.zeros((L,), jnp.float32)

        for cc in range(n_chunks):
            row0 = pl.multiple_of(wid * rows_per_w + cc * SLAB, 8)
            pltpu.sync_copy(u_hbm.at[pl.ds(row0, SLAB)], u_v)
            pltpu.sync_copy(v_hbm.at[pl.ds(row0, SLAB)], v_v)

            def group_body(g, _, cc=cc):
                res = zeros
                for r in range(L):
                    i = g * L + r
                    acc = (u_v[i, pl.ds(0, L)] * v_v[i, pl.ds(0, L)]) * w[0]
                    for q in range(1, FACTORS // L):
                        acc += (u_v[i, pl.ds(q * L, L)]
                                * v_v[i, pl.ds(q * L, L)]) * w[q]
                    cum = plsc.cumsum(acc)
                    total = _bcast_dyn(cum, L - 1)
                    res = jnp.where(lane_iota == r, total, res)
                res = jnp.maximum(res + bias, 0.0)
                out_v[pl.ds(cc * SLAB + g * L, L)] = res
                return 0

            lax.fori_loop(0, SLAB // L, group_body, 0)

        pltpu.sync_copy(out_v, out_hbm.at[wid])

    return k(u_rows, v_rows, wb)


def _schedule(idx):
    srt = jnp.sort(idx).reshape(NS, ITEMS_PER_TILE)
    pos = jnp.argsort(idx).astype(jnp.int32).reshape(NS, ITEMS_PER_TILE)
    seg = srt >> 7
    prev = jnp.concatenate(
        [jnp.full((NS, 1), -1, jnp.int32), seg[:, :-1]], axis=1)
    new = (seg != prev).astype(jnp.int32)
    ordt = jnp.cumsum(new, axis=1) - 1
    rows = jnp.broadcast_to(
        jnp.arange(NS, dtype=jnp.int32)[:, None], seg.shape)
    slabs = jnp.zeros((NS, ITEMS_PER_TILE), jnp.int32).at[rows, ordt].set(seg)
    shp = (NS, GROUPS, SLAB)
    return (srt.reshape(shp), pos.reshape(shp), ordt.reshape(shp),
            slabs.reshape(shp))


def kernel(user_coordinates, item_coordinates, user_table, item_table, W1, b1):
    batch = user_coordinates.shape[0]
    uidx = user_coordinates.astype(jnp.int32)
    iidx = item_coordinates.astype(jnp.int32)
    su, pu, ou, lu = _schedule(uidx)
    si, pi, oi, li = _schedule(iidx)
    wb = jnp.concatenate(
        [W1.reshape(-1), b1.reshape(-1),
         jnp.zeros((WB_PAD - FACTORS - 1,), jnp.float32)])
    u_rows, v_rows = _gather_phase(
        su, pu, ou, lu, si, pi, oi, li, user_table.T, item_table.T)
    out = _dot_phase(u_rows, v_rows, wb)
    return out.reshape(batch, 1)


# speedup vs baseline: 3.0768x; 1.0759x over previous
import functools

import jax
import jax.numpy as jnp
from jax import lax
from jax.experimental import pallas as pl
from jax.experimental.pallas import tpu as pltpu
from jax.experimental.pallas import tpu_sc as plsc

FACTORS = 64
L = 16
NC = 2
NS = 16
NW = NC * NS
SLAB = 128
BATCH = 16384
ITEMS_PER_TILE = BATCH // NS
GROUPS = ITEMS_PER_TILE // SLAB
ROWS_PAD = 128
WB_PAD = 96


def _bcast_lane0(vec):
    idx = jnp.zeros((L, 1), jnp.int32)
    dn = lax.GatherDimensionNumbers(
        offset_dims=(), collapsed_slice_dims=(0,), start_index_map=(0,))
    return lax.gather(vec, idx, dn, (1,),
                      mode=lax.GatherScatterMode.PROMISE_IN_BOUNDS)


def _bcast_dyn(vec, lane):
    idx = jnp.full((L, 1), lane, jnp.int32)
    dn = lax.GatherDimensionNumbers(
        offset_dims=(), collapsed_slice_dims=(0,), start_index_map=(0,))
    return lax.gather(vec, idx, dn, (1,),
                      mode=lax.GatherScatterMode.PROMISE_IN_BOUNDS)


NBUF = 8
LOOKAHEAD = 7


def _gather_phase(su_u, pu_u, ou_u, lu_u, si_i, pi_i, oi_i, li_i,
                  utab_t, itab_t):
    mesh = plsc.VectorSubcoreMesh(core_axis_name="c", subcore_axis_name="s")

    @functools.partial(
        pl.kernel,
        mesh=mesh,
        out_type=(
            jax.ShapeDtypeStruct((BATCH, ROWS_PAD), jnp.float32),
            jax.ShapeDtypeStruct((BATCH, ROWS_PAD), jnp.float32),
        ),
        scratch_types=[
            pltpu.VMEM((GROUPS, SLAB), jnp.int32),
            pltpu.VMEM((GROUPS, SLAB), jnp.int32),
            pltpu.VMEM((GROUPS, SLAB), jnp.int32),
            pltpu.VMEM((GROUPS, SLAB), jnp.int32),
            pltpu.VMEM((NBUF, FACTORS, SLAB), jnp.float32),
            pltpu.VMEM((2, SLAB, ROWS_PAD), jnp.float32),
        ] + [pltpu.SemaphoreType.DMA] * (NBUF + 1),
        compiler_params=pltpu.CompilerParams(
            needs_layout_passes=False, use_tc_tiling_on_sc=True),
    )
    def k(su_ref, pu_ref, ou_ref, lu_ref, si_ref, pi_ref, oi_ref, li_ref,
          ut_ref, it_ref, u_out, v_out,
          srt_v, pos_v, ord_v, slabs_v, ring_v, ext_v, *sems_all):
        c = lax.axis_index("c")
        s = lax.axis_index("s")
        sems = list(sems_all[:NBUF])
        scat_sem = sems_all[NBUF]
        lane_iota = lax.iota(jnp.int32, L)

        def side(tab, srt_hbm, pos_hbm, ordh, slabh, out_hbm):
            pltpu.sync_copy(srt_hbm.at[s], srt_v)
            pltpu.sync_copy(pos_hbm.at[s], pos_v)
            pltpu.sync_copy(ordh.at[s], ord_v)
            pltpu.sync_copy(slabh.at[s], slabs_v)

            def fire(b, p):
                pc = jnp.minimum(p, GROUPS * SLAB - 1)
                prow = pc >> 7
                pcb = ((pc & 127) >> 4) << 4
                pchunk = slabs_v[prow, pl.ds(pl.multiple_of(pcb, 8), L)]
                sid = _bcast_dyn(pchunk, pc & 15)[0]
                off = pl.multiple_of(sid * SLAB, SLAB)
                pltpu.async_copy(tab.at[:, pl.ds(off, SLAB)],
                                 ring_v.at[b], sems[b])

            def drain(b):
                pltpu.make_async_copy(tab.at[:, pl.ds(0, SLAB)],
                                      ring_v.at[b], sems[b]).wait()

            for b in range(LOOKAHEAD):
                fire(b, jnp.int32(b))

            prev = jnp.int32(-1)
            pending = [None, None]
            for g in range(GROUPS):
                if pending[g & 1] is not None:
                    pending[g & 1].wait()
                    pending[g & 1] = None
                def body(i, prev, g=g):
                    chunk_base = (i >> 4) << 4
                    chunk = srt_v[g, pl.ds(pl.multiple_of(chunk_base, 8), L)]
                    j = i & 15
                    clv = _bcast_dyn(chunk & (SLAB - 1), j)
                    ochunk = ord_v[g, pl.ds(pl.multiple_of(chunk_base, 8), L)]
                    odv = _bcast_dyn(ochunk, j)
                    od = odv[0]

                    @pl.when(od != prev)
                    def _():
                        for b in range(NBUF):
                            @pl.when((od & (NBUF - 1)) == b)
                            def _(b=b):
                                drain(b)
                                fire((b + LOOKAHEAD) % NBUF,
                                     od + LOOKAHEAD)

                    bsel = odv & (NBUF - 1)
                    for q in range(FACTORS // L):
                        vec = plsc.load_gather(
                            ring_v, [bsel, lane_iota + q * L, clv])
                        ext_v[g & 1, i, pl.ds(q * L, L)] = vec
                    return od

                prev = lax.fori_loop(0, SLAB, body, prev)
                pending[g & 1] = pltpu.async_copy(
                    ext_v.at[g & 1], out_hbm.at[pos_v.at[g]], scat_sem)
            for h in pending:
                if h is not None:
                    h.wait()

            for r in range(NBUF):
                @pl.when((prev & (NBUF - 1)) == r)
                def _(r=r):
                    for d in range(1, LOOKAHEAD + 1):
                        drain((r + d) % NBUF)

        @pl.when(c == 0)
        def _():
            side(ut_ref, su_ref, pu_ref, ou_ref, lu_ref, u_out)

        @pl.when(c == 1)
        def _():
            side(it_ref, si_ref, pi_ref, oi_ref, li_ref, v_out)

    return k(su_u, pu_u, ou_u, lu_u, si_i, pi_i, oi_i, li_i, utab_t, itab_t)


def _dot_phase(u_rows, v_rows, wb):
    mesh = plsc.VectorSubcoreMesh(core_axis_name="c", subcore_axis_name="s")
    rows_per_w = BATCH // NW
    n_chunks = rows_per_w // SLAB

    @functools.partial(
        pl.kernel,
        mesh=mesh,
        out_type=jax.ShapeDtypeStruct((NW, rows_per_w), jnp.float32),
        scratch_types=[
            pltpu.VMEM((SLAB, ROWS_PAD), jnp.float32),
            pltpu.VMEM((SLAB, ROWS_PAD), jnp.float32),
            pltpu.VMEM((WB_PAD,), jnp.float32),
            pltpu.VMEM((rows_per_w,), jnp.float32),
        ],
        compiler_params=pltpu.CompilerParams(
            needs_layout_passes=False, use_tc_tiling_on_sc=True),
    )
    def k(u_hbm, v_hbm, wb_hbm, out_hbm, u_v, v_v, wb_v, out_v):
        wid = lax.axis_index("s") * NC + lax.axis_index("c")
        pltpu.sync_copy(wb_hbm, wb_v)
        w = [wb_v[pl.ds(q * L, L)] for q in range(FACTORS // L)]
        bias = _bcast_lane0(wb_v[pl.ds(FACTORS, L)])
        lane_iota = lax.iota(jnp.int32, L)
        zeros = jnp.zeros((L,), jnp.float32)

        for cc in range(n_chunks):
            row0 = pl.multiple_of(wid * rows_per_w + cc * SLAB, 8)
            pltpu.sync_copy(u_hbm.at[pl.ds(row0, SLAB)], u_v)
            pltpu.sync_copy(v_hbm.at[pl.ds(row0, SLAB)], v_v)

            def group_body(g, _, cc=cc):
                res = zeros
                for r in range(L):
                    i = g * L + r
                    acc = (u_v[i, pl.ds(0, L)] * v_v[i, pl.ds(0, L)]) * w[0]
                    for q in range(1, FACTORS // L):
                        acc += (u_v[i, pl.ds(q * L, L)]
                                * v_v[i, pl.ds(q * L, L)]) * w[q]
                    cum = plsc.cumsum(acc)
                    total = _bcast_dyn(cum, L - 1)
                    res = jnp.where(lane_iota == r, total, res)
                res = jnp.maximum(res + bias, 0.0)
                out_v[pl.ds(cc * SLAB + g * L, L)] = res
                return 0

            lax.fori_loop(0, SLAB // L, group_body, 0)

        pltpu.sync_copy(out_v, out_hbm.at[wid])

    return k(u_rows, v_rows, wb)


def _schedule(idx):
    srt = jnp.sort(idx).reshape(NS, ITEMS_PER_TILE)
    pos = jnp.argsort(idx).astype(jnp.int32).reshape(NS, ITEMS_PER_TILE)
    seg = srt >> 7
    prev = jnp.concatenate(
        [jnp.full((NS, 1), -1, jnp.int32), seg[:, :-1]], axis=1)
    new = (seg != prev).astype(jnp.int32)
    ordt = jnp.cumsum(new, axis=1) - 1
    rows = jnp.broadcast_to(
        jnp.arange(NS, dtype=jnp.int32)[:, None], seg.shape)
    slabs = jnp.zeros((NS, ITEMS_PER_TILE), jnp.int32).at[rows, ordt].set(seg)
    shp = (NS, GROUPS, SLAB)
    return (srt.reshape(shp), pos.reshape(shp), ordt.reshape(shp),
            slabs.reshape(shp))


def kernel(user_coordinates, item_coordinates, user_table, item_table, W1, b1):
    batch = user_coordinates.shape[0]
    uidx = user_coordinates.astype(jnp.int32)
    iidx = item_coordinates.astype(jnp.int32)
    su, pu, ou, lu = _schedule(uidx)
    si, pi, oi, li = _schedule(iidx)
    wb = jnp.concatenate(
        [W1.reshape(-1), b1.reshape(-1),
         jnp.zeros((WB_PAD - FACTORS - 1,), jnp.float32)])
    u_rows, v_rows = _gather_phase(
        su, pu, ou, lu, si, pi, oi, li, user_table.T, item_table.T)
    out = _dot_phase(u_rows, v_rows, wb)
    return out.reshape(batch, 1)
